# Initial kernel scaffold; baseline (speedup 1.0000x reference)
import jax
import jax.numpy as jnp
from jax.experimental import pallas as pl


def kernel(loss):
    raise NotImplementedError("write your pallas kernel here")


# speedup vs baseline: 30.4262x; 30.4262x over previous
import functools

import jax
import jax.numpy as jnp
from jax.experimental import pallas as pl

_PERC = 0.25


def _topk_mean_body(k, inv_total, x_ref, o_ref):
    int_min = jnp.int32(-(2**31))
    x = x_ref[...]
    bits = jax.lax.bitcast_convert_type(x, jnp.int32)
    key = jnp.where(bits >= 0, bits, int_min - bits)

    kf = jnp.float32(k)

    def bit_step(i, prefix_u):
        bit = jnp.int32(31) - i
        cand_u = prefix_u | (jnp.int32(1) << bit)
        cand_s = cand_u ^ int_min
        cnt = jnp.sum((key >= cand_s).astype(jnp.float32), axis=1, keepdims=True)
        return jnp.where(cnt >= kf, cand_u, prefix_u)

    prefix_u = jnp.zeros((x.shape[0], 1), jnp.int32)
    t_u = jax.lax.fori_loop(0, 32, bit_step, prefix_u)
    t_s = t_u ^ int_min

    gt = key > t_s
    cnt_gt = jnp.sum(gt.astype(jnp.float32), axis=1, keepdims=True)
    sum_gt = jnp.sum(jnp.where(gt, x, 0.0), axis=1, keepdims=True)
    t_f = jax.lax.bitcast_convert_type(
        jnp.where(t_s >= 0, t_s, int_min - t_s), jnp.float32
    )
    row_sum = sum_gt + (kf - cnt_gt) * t_f
    o_ref[...] = jnp.sum(row_sum).reshape(1, 1) * inv_total


def kernel(loss):
    B = loss.shape[0]
    loss2 = loss.reshape(B, -1)
    P = loss2.shape[1]
    k = int(_PERC * P)
    out = pl.pallas_call(
        functools.partial(_topk_mean_body, k, 1.0 / (B * k)),
        out_shape=jax.ShapeDtypeStruct((1, 1), jnp.float32),
    )(loss2)
    return out[0, 0]
